# baseline (speedup 1.0000x reference)
import functools

import jax
import jax.numpy as jnp
from jax import lax
from jax.experimental import pallas as pl
from jax.experimental.pallas import tpu as pltpu
from jax.experimental.pallas import tpu_sc as plsc

B, S, D = 4, 8192, 4096
NW = 32
POS_PER_W = S // NW
CH = 8
NCHUNK = POS_PER_W // CH
VECS = CH * D // 16
UNROLL = 8
TC_B = 3


def _tc_body(seq_ref, tab_ref, out_ref):
    out_ref[...] = seq_ref[...] + tab_ref[...][None, :, :]


def _tc_add(seq, pos_table):
    CHUNK = 128
    grid = (S // CHUNK,)
    return pl.pallas_call(
        _tc_body,
        grid=grid,
        in_specs=[
            pl.BlockSpec((TC_B, CHUNK, D), lambda i: (0, i, 0)),
            pl.BlockSpec((CHUNK, D), lambda i: (i, 0)),
        ],
        out_specs=pl.BlockSpec((TC_B, CHUNK, D), lambda i: (0, i, 0)),
        out_shape=jax.ShapeDtypeStruct((TC_B, S, D), seq.dtype),
        compiler_params=pltpu.CompilerParams(
            dimension_semantics=("parallel",),
        ),
    )(seq, pos_table)


def _sc_body(seq_hbm, tab_hbm, out_hbm, tbuf, sbuf):
    wid = lax.axis_index("s") * 2 + lax.axis_index("c")
    base = wid * POS_PER_W

    def chunk(c, _):
        s0 = base + c * CH
        pltpu.sync_copy(tab_hbm.at[pl.ds(s0, CH), :], tbuf)
        pltpu.sync_copy(seq_hbm.at[TC_B, pl.ds(s0, CH), :], sbuf)

        def add(i, _):
            for k in range(UNROLL):
                r = i * UNROLL + k
                row = r // (D // 16)
                col = (r % (D // 16)) * 16
                sbuf[row, pl.ds(col, 16)] = (
                    sbuf[row, pl.ds(col, 16)] + tbuf[row, pl.ds(col, 16)]
                )
            return 0

        lax.fori_loop(0, VECS // UNROLL, add, 0)
        pltpu.sync_copy(sbuf, out_hbm.at[0, pl.ds(s0, CH), :])
        return 0

    lax.fori_loop(0, NCHUNK, chunk, 0)


def _sc_add(seq, tab):
    mesh = plsc.VectorSubcoreMesh(core_axis_name="c", subcore_axis_name="s")
    return functools.partial(
        pl.kernel,
        mesh=mesh,
        out_type=jax.ShapeDtypeStruct((B - TC_B, S, D), jnp.float32),
        scratch_types=[
            pltpu.VMEM((CH, D), jnp.float32),
            pltpu.VMEM((CH, D), jnp.float32),
        ],
        compiler_params=pltpu.CompilerParams(use_tc_tiling_on_sc=True),
    )(_sc_body)(seq, tab)


@jax.jit
def _pos_add(seq, tab):
    sc_out = _sc_add(seq, tab)
    tc_out = _tc_add(seq, tab)
    return jnp.concatenate([tc_out, sc_out], axis=0)


def kernel(seq, pos_table):
    s = seq.shape[1]
    return _pos_add(seq, pos_table[:s, :])
